# baseline (speedup 1.0000x reference)
import functools
import jax
import jax.numpy as jnp
from jax import lax
from jax.experimental import pallas as pl
from jax.experimental.pallas import tpu as pltpu
from jax.experimental.pallas import tpu_sc as plsc

_N = 1000
_E = 100
_NP = 1024
_EP = 128

_NC = 2
_NS = 16
_NW = _NC * _NS
_ROWS = _NP // _NW


def _sc_build_operators(edge_hbm, ps_hbm, pd_hbm, src_v, dst_v, slab):
    wid = lax.axis_index("s") * _NC + lax.axis_index("c")
    base = wid * _ROWS
    slab_elems = _ROWS * _EP
    pltpu.sync_copy(edge_hbm.at[0], src_v)
    pltpu.sync_copy(edge_hbm.at[1], dst_v)

    one = jnp.ones((16,), jnp.float32)
    zero = jnp.zeros((16,), jnp.float32)

    svec = [src_v[pl.ds(j * 16, 16)] for j in range(_EP // 16)]
    for r in range(_ROWS):
        node = base + r
        for j in range(_EP // 16):
            slab[pl.ds(r * _EP + j * 16, 16)] = jnp.where(svec[j] == node, one, zero)
    pltpu.sync_copy(slab, ps_hbm.at[pl.ds(base * _EP, slab_elems)])

    dvec = [dst_v[pl.ds(j * 16, 16)] for j in range(_EP // 16)]
    for r in range(_ROWS):
        node = base + r
        for j in range(_EP // 16):
            slab[pl.ds(r * _EP + j * 16, 16)] = jnp.where(dvec[j] == node, one, zero)
    pltpu.sync_copy(slab, pd_hbm.at[pl.ds(base * _EP, slab_elems)])


_sc_prep = functools.partial(
    pl.kernel,
    out_type=(
        jax.ShapeDtypeStruct((_NP * _EP,), jnp.float32),
        jax.ShapeDtypeStruct((_NP * _EP,), jnp.float32),
    ),
    mesh=plsc.VectorSubcoreMesh(core_axis_name="c", subcore_axis_name="s"),
    scratch_types=[
        pltpu.VMEM((_EP,), jnp.int32),
        pltpu.VMEM((_EP,), jnp.int32),
        pltpu.VMEM((_ROWS * _EP,), jnp.float32),
    ],
)(_sc_build_operators)


def _fused_body(ps_ref, pd_ref, x_ref,
                wl0, bl0, wr0, wl1, bl1, wr1, wl2, bl2, wr2,
                wl3, bl3, wr3, wl4, bl4, wr4, out_ref):
    p_src_t = ps_ref[...]
    p_dst = pd_ref[...]
    count = jnp.sum(p_dst, axis=1, keepdims=True)
    p_dst = p_dst / jnp.maximum(count, 1.0)

    weights = ((wl0, bl0, wr0), (wl1, bl1, wr1), (wl2, bl2, wr2),
               (wl3, bl3, wr3), (wl4, bl4, wr4))

    h = x_ref[...]
    for i, (wl, bl, wr) in enumerate(weights):
        if i > 0:
            h = jnp.maximum(h, 0.0)
        xj = lax.dot_general(p_src_t, h, (((0,), (0,)), ((), ())),
                             preferred_element_type=jnp.float32)
        m = lax.dot_general(xj, wl[...], (((1,), (1,)), ((), ())),
                            preferred_element_type=jnp.float32)
        aggl = lax.dot_general(p_dst, m, (((1,), (0,)), ((), ())),
                               preferred_element_type=jnp.float32)
        dense = lax.dot_general(h, wr[...], (((1,), (1,)), ((), ())),
                                preferred_element_type=jnp.float32)
        h = aggl + dense + bl[...]
    out_ref[...] = h


def kernel(L_x_, L_edge_index_, L_self_modules_convs_modules_0_modules_lin_l_parameters_weight_, L_self_modules_convs_modules_0_modules_lin_l_parameters_bias_, L_self_modules_convs_modules_0_modules_lin_r_parameters_weight_, L_self_modules_convs_modules_1_modules_lin_l_parameters_weight_, L_self_modules_convs_modules_1_modules_lin_l_parameters_bias_, L_self_modules_convs_modules_1_modules_lin_r_parameters_weight_, L_self_modules_convs_modules_2_modules_lin_l_parameters_weight_, L_self_modules_convs_modules_2_modules_lin_l_parameters_bias_, L_self_modules_convs_modules_2_modules_lin_r_parameters_weight_, L_self_modules_convs_modules_3_modules_lin_l_parameters_weight_, L_self_modules_convs_modules_3_modules_lin_l_parameters_bias_, L_self_modules_convs_modules_3_modules_lin_r_parameters_weight_, L_self_modules_convs_modules_4_modules_lin_l_parameters_weight_, L_self_modules_convs_modules_4_modules_lin_l_parameters_bias_, L_self_modules_convs_modules_4_modules_lin_r_parameters_weight_):
    x = L_x_
    edge = L_edge_index_
    edge_p = jnp.full((2, _EP), -1, dtype=jnp.int32).at[:, :_E].set(edge)
    x_p = jnp.zeros((_NP, 128), dtype=jnp.float32).at[:_N, :].set(x)

    ps_flat, pd_flat = _sc_prep(edge_p)
    p_src_t = ps_flat.reshape(_NP, _EP)
    p_dst_raw = pd_flat.reshape(_NP, _EP)

    ws = [
        L_self_modules_convs_modules_0_modules_lin_l_parameters_weight_,
        L_self_modules_convs_modules_0_modules_lin_l_parameters_bias_.reshape(1, -1),
        L_self_modules_convs_modules_0_modules_lin_r_parameters_weight_,
        L_self_modules_convs_modules_1_modules_lin_l_parameters_weight_,
        L_self_modules_convs_modules_1_modules_lin_l_parameters_bias_.reshape(1, -1),
        L_self_modules_convs_modules_1_modules_lin_r_parameters_weight_,
        L_self_modules_convs_modules_2_modules_lin_l_parameters_weight_,
        L_self_modules_convs_modules_2_modules_lin_l_parameters_bias_.reshape(1, -1),
        L_self_modules_convs_modules_2_modules_lin_r_parameters_weight_,
        L_self_modules_convs_modules_3_modules_lin_l_parameters_weight_,
        L_self_modules_convs_modules_3_modules_lin_l_parameters_bias_.reshape(1, -1),
        L_self_modules_convs_modules_3_modules_lin_r_parameters_weight_,
        L_self_modules_convs_modules_4_modules_lin_l_parameters_weight_,
        L_self_modules_convs_modules_4_modules_lin_l_parameters_bias_.reshape(1, -1),
        L_self_modules_convs_modules_4_modules_lin_r_parameters_weight_,
    ]

    out = pl.pallas_call(
        _fused_body,
        out_shape=jax.ShapeDtypeStruct((_NP, 256), jnp.float32),
    )(p_src_t, p_dst_raw, x_p, *ws)
    return out[:_N]


# speedup vs baseline: 1.0530x; 1.0530x over previous
import functools
import jax
import jax.numpy as jnp
from jax import lax
from jax.experimental import pallas as pl
from jax.experimental.pallas import tpu as pltpu
from jax.experimental.pallas import tpu_sc as plsc

_N = 1000
_E = 100
_NP = 1024
_EP = 128

_NC = 2
_NS = 16
_NW = _NC * _NS
_ROWS = _NP // _NW


def _sc_build_operators(edge_hbm, ps_hbm, pd_hbm, src_v, dst_v, slab):
    wid = lax.axis_index("s") * _NC + lax.axis_index("c")
    base = wid * _ROWS
    slab_elems = _ROWS * _EP
    pltpu.sync_copy(edge_hbm.at[0], src_v)
    pltpu.sync_copy(edge_hbm.at[1], dst_v)

    one = jnp.ones((16,), jnp.float32)
    zero = jnp.zeros((16,), jnp.float32)

    svec = [src_v[pl.ds(j * 16, 16)] for j in range(_EP // 16)]
    for r in range(_ROWS):
        node = base + r
        for j in range(_EP // 16):
            slab[pl.ds(r * _EP + j * 16, 16)] = jnp.where(svec[j] == node, one, zero)
    pltpu.sync_copy(slab, ps_hbm.at[pl.ds(base * _EP, slab_elems)])

    dvec = [dst_v[pl.ds(j * 16, 16)] for j in range(_EP // 16)]
    for r in range(_ROWS):
        node = base + r
        for j in range(_EP // 16):
            slab[pl.ds(r * _EP + j * 16, 16)] = jnp.where(dvec[j] == node, one, zero)
    pltpu.sync_copy(slab, pd_hbm.at[pl.ds(base * _EP, slab_elems)])


_sc_prep = functools.partial(
    pl.kernel,
    out_type=(
        jax.ShapeDtypeStruct((_NP * _EP,), jnp.float32),
        jax.ShapeDtypeStruct((_NP * _EP,), jnp.float32),
    ),
    mesh=plsc.VectorSubcoreMesh(core_axis_name="c", subcore_axis_name="s"),
    scratch_types=[
        pltpu.VMEM((_EP,), jnp.int32),
        pltpu.VMEM((_EP,), jnp.int32),
        pltpu.VMEM((_ROWS * _EP,), jnp.float32),
    ],
)(_sc_build_operators)


def _tc_pre_body(x_ref, wl0_ref, wr0_ref, hm_ref, d_ref):
    x = jnp.pad(x_ref[...], ((0, _NP - _N), (0, 0)))
    hm_ref[...] = lax.dot_general(x, wl0_ref[...], (((1,), (1,)), ((), ())),
                                  preferred_element_type=jnp.float32)
    d_ref[...] = lax.dot_general(x, wr0_ref[...], (((1,), (1,)), ((), ())),
                                 preferred_element_type=jnp.float32)


def _fused_body(ps_ref, pd_ref, hm0_ref, d0_ref, bl0,
                wl1, bl1, wr1, wl2, bl2, wr2,
                wl3, bl3, wr3, wl4, bl4, wr4, out_ref):
    p_src_t = ps_ref[...]
    p_dst = pd_ref[...]
    count = jnp.sum(p_dst, axis=1, keepdims=True)
    p_dst = p_dst / jnp.maximum(count, 1.0)

    m0 = lax.dot_general(p_src_t, hm0_ref[...], (((0,), (0,)), ((), ())),
                         preferred_element_type=jnp.float32)
    aggl0 = lax.dot_general(p_dst, m0, (((1,), (0,)), ((), ())),
                            preferred_element_type=jnp.float32)
    h = aggl0 + d0_ref[...] + bl0[...]

    weights = ((wl1, bl1, wr1), (wl2, bl2, wr2), (wl3, bl3, wr3), (wl4, bl4, wr4))
    for wl, bl, wr in weights:
        h = jnp.maximum(h, 0.0)
        xj = lax.dot_general(p_src_t, h, (((0,), (0,)), ((), ())),
                             preferred_element_type=jnp.float32)
        m = lax.dot_general(xj, wl[...], (((1,), (1,)), ((), ())),
                            preferred_element_type=jnp.float32)
        aggl = lax.dot_general(p_dst, m, (((1,), (0,)), ((), ())),
                               preferred_element_type=jnp.float32)
        dense = lax.dot_general(h, wr[...], (((1,), (1,)), ((), ())),
                                preferred_element_type=jnp.float32)
        h = aggl + dense + bl[...]
    out_ref[...] = h[:_N]


def kernel(L_x_, L_edge_index_, L_self_modules_convs_modules_0_modules_lin_l_parameters_weight_, L_self_modules_convs_modules_0_modules_lin_l_parameters_bias_, L_self_modules_convs_modules_0_modules_lin_r_parameters_weight_, L_self_modules_convs_modules_1_modules_lin_l_parameters_weight_, L_self_modules_convs_modules_1_modules_lin_l_parameters_bias_, L_self_modules_convs_modules_1_modules_lin_r_parameters_weight_, L_self_modules_convs_modules_2_modules_lin_l_parameters_weight_, L_self_modules_convs_modules_2_modules_lin_l_parameters_bias_, L_self_modules_convs_modules_2_modules_lin_r_parameters_weight_, L_self_modules_convs_modules_3_modules_lin_l_parameters_weight_, L_self_modules_convs_modules_3_modules_lin_l_parameters_bias_, L_self_modules_convs_modules_3_modules_lin_r_parameters_weight_, L_self_modules_convs_modules_4_modules_lin_l_parameters_weight_, L_self_modules_convs_modules_4_modules_lin_l_parameters_bias_, L_self_modules_convs_modules_4_modules_lin_r_parameters_weight_):
    x = L_x_
    edge = L_edge_index_.astype(jnp.int32)
    edge_p = jnp.pad(edge, ((0, 0), (0, _EP - _E)), constant_values=-1)

    ps_flat, pd_flat = _sc_prep(edge_p)
    p_src_t = ps_flat.reshape(_NP, _EP)
    p_dst_raw = pd_flat.reshape(_NP, _EP)

    hm0, d0 = pl.pallas_call(
        _tc_pre_body,
        out_shape=(
            jax.ShapeDtypeStruct((_NP, 256), jnp.float32),
            jax.ShapeDtypeStruct((_NP, 256), jnp.float32),
        ),
    )(x,
      L_self_modules_convs_modules_0_modules_lin_l_parameters_weight_,
      L_self_modules_convs_modules_0_modules_lin_r_parameters_weight_)

    ws = [
        L_self_modules_convs_modules_0_modules_lin_l_parameters_bias_.reshape(1, -1),
        L_self_modules_convs_modules_1_modules_lin_l_parameters_weight_,
        L_self_modules_convs_modules_1_modules_lin_l_parameters_bias_.reshape(1, -1),
        L_self_modules_convs_modules_1_modules_lin_r_parameters_weight_,
        L_self_modules_convs_modules_2_modules_lin_l_parameters_weight_,
        L_self_modules_convs_modules_2_modules_lin_l_parameters_bias_.reshape(1, -1),
        L_self_modules_convs_modules_2_modules_lin_r_parameters_weight_,
        L_self_modules_convs_modules_3_modules_lin_l_parameters_weight_,
        L_self_modules_convs_modules_3_modules_lin_l_parameters_bias_.reshape(1, -1),
        L_self_modules_convs_modules_3_modules_lin_r_parameters_weight_,
        L_self_modules_convs_modules_4_modules_lin_l_parameters_weight_,
        L_self_modules_convs_modules_4_modules_lin_l_parameters_bias_.reshape(1, -1),
        L_self_modules_convs_modules_4_modules_lin_r_parameters_weight_,
    ]

    out = pl.pallas_call(
        _fused_body,
        out_shape=jax.ShapeDtypeStruct((_N, 256), jnp.float32),
    )(p_src_t, p_dst_raw, hm0, d0, *ws)
    return out
